# baseline (speedup 1.0000x reference)
import functools

import jax
import jax.numpy as jnp
from jax import lax
from jax.experimental import pallas as pl
from jax.experimental.pallas import tpu as pltpu
from jax.experimental.pallas import tpu_sc as plsc

NC, NS, LANES = 2, 16, 16
CH = 80
ZR = 16
DEGW = 16


def _npad(n):
    step = NS * 8
    return ((n + step - 1) // step) * step


def _sc_mesh():
    return plsc.VectorSubcoreMesh(
        core_axis_name="c", subcore_axis_name="s",
        num_cores=NC, num_subcores=NS)


def _make_deg_kernel(n, e):
    nw = NC * NS
    e_tile = e // nw
    npd = _npad(n)

    @functools.partial(
        pl.kernel,
        out_type=jax.ShapeDtypeStruct((nw, npd), jnp.float32),
        mesh=_sc_mesh(),
        compiler_params=pltpu.CompilerParams(needs_layout_passes=False),
        scratch_types=[
            pltpu.VMEM((e_tile,), jnp.int32),
            pltpu.VMEM((npd,), jnp.float32),
        ])
    def deg_kernel(dst_hbm, out_hbm, idx_v, hist_v):
        c = lax.axis_index("c")
        s = lax.axis_index("s")
        wid = c * NS + s
        base = wid * e_tile
        pltpu.sync_copy(dst_hbm.at[pl.ds(base, e_tile)], idx_v)

        def initz(i, carry):
            hist_v[pl.ds(i * LANES, LANES)] = jnp.zeros((LANES,),
                                                        jnp.float32)
            return carry
        lax.fori_loop(0, npd // LANES, initz, None)
        ones = jnp.ones((LANES,), jnp.float32)

        def body(i, carry):
            idx = idx_v[pl.ds(i * LANES, LANES)]
            plsc.addupdate_scatter(hist_v, [idx], ones)
            return carry
        lax.fori_loop(0, e_tile // LANES, body, None)
        pltpu.sync_copy(hist_v, out_hbm.at[wid])

    return deg_kernel


_NB = 7
_ACH = 40


def _make_agg_kernel(n, e, d):
    e_sc, e_tile = e // NC, e // (NC * NS)
    nch = e_tile // _ACH
    ngroups = nch // _NB
    npd = _npad(n)
    rpt = npd // NS
    nz = rpt // ZR
    nvec = d // LANES

    @functools.partial(
        pl.kernel,
        out_type=jax.ShapeDtypeStruct((NC, npd, d), jnp.float32),
        mesh=_sc_mesh(),
        scratch_types=[
            pltpu.VMEM((e_tile,), jnp.int32),
            [pltpu.VMEM((_ACH,), jnp.int32) for _ in range(_NB)],
            [pltpu.VMEM((_ACH, d), jnp.float32) for _ in range(_NB)],
            pltpu.VMEM((ZR, d), jnp.float32),
            pltpu.VMEM_SHARED((npd, d), jnp.float32),
            pltpu.SemaphoreType.DMA((_NB,)),
            pltpu.SemaphoreType.DMA((_NB,)),
            pltpu.SemaphoreType.DMA,
        ])
    def agg_kernel(y_hbm, src_hbm, dst_hbm, out_hbm,
                   srcall_v, dring, rows, zero_v, acc_sh, dsem, gsem,
                   zsem):
        c = lax.axis_index("c")
        s = lax.axis_index("s")
        base = c * e_sc + s * e_tile

        pltpu.sync_copy(src_hbm.at[pl.ds(base, e_tile)], srcall_v)

        def issue(j, b):
            off = pl.multiple_of(base + j * _ACH, 8)
            pltpu.async_copy(dst_hbm.at[pl.ds(off, _ACH)], dring[b],
                             dsem.at[b])
            idx = srcall_v.at[pl.ds(j * _ACH, _ACH)]
            pltpu.async_copy(y_hbm.at[idx], rows[b], gsem.at[b])

        def drain_and_scatter(j, b):
            off = pl.multiple_of(base + j * _ACH, 8)
            pltpu.make_async_copy(dst_hbm.at[pl.ds(off, _ACH)], dring[b],
                                  dsem.at[b]).wait()
            idx = srcall_v.at[pl.ds(j * _ACH, _ACH)]
            pltpu.make_async_copy(y_hbm.at[idx], rows[b],
                                  gsem.at[b]).wait()
            pltpu.sync_copy(rows[b], acc_sh.at[dring[b]], add=True)

        for b in range(_NB):
            issue(b, b)

        def initz(i, carry):
            for q in range(nvec):
                zero_v[i, pl.ds(q * LANES, LANES)] = (
                    jnp.zeros((LANES,), jnp.float32))
            return carry
        lax.fori_loop(0, ZR, initz, None)

        r0 = s * rpt
        for q in range(nz):
            pltpu.async_copy(zero_v, acc_sh.at[pl.ds(r0 + q * ZR, ZR)],
                             zsem)
        for q in range(nz):
            pltpu.make_async_copy(zero_v,
                                  acc_sh.at[pl.ds(r0 + q * ZR, ZR)],
                                  zsem).wait()
        plsc.subcore_barrier()

        def group(t, carry):
            j0 = t * _NB
            for b in range(_NB):
                drain_and_scatter(j0 + b, b)
                issue(j0 + b + _NB, b)
            return carry
        lax.fori_loop(0, ngroups - 1, group, None)
        for b in range(_NB):
            drain_and_scatter((ngroups - 1) * _NB + b, b)
        for j in range(ngroups * _NB, nch):
            issue(j, 0)
            drain_and_scatter(j, 0)

        plsc.subcore_barrier()
        pltpu.sync_copy(acc_sh.at[pl.ds(r0, rpt)],
                        out_hbm.at[c, pl.ds(r0, rpt)])

    return agg_kernel


_BR = 1024


def _head_body(x_ref, w_ref, degp_ref, y_ref, dinv_ref):
    deg = jnp.sum(degp_ref[...], axis=0).reshape(-1, 1) + 1.0
    dinv = lax.rsqrt(deg)
    dinv_ref[...] = dinv
    y_ref[...] = dinv * jnp.dot(x_ref[...], w_ref[...],
                                preferred_element_type=jnp.float32)


def _head(x, w, degp):
    n, k = x.shape
    m = w.shape[1]
    npd = degp.shape[1]
    return pl.pallas_call(
        _head_body,
        grid=(pl.cdiv(npd, _BR),),
        in_specs=[pl.BlockSpec((_BR, k), lambda i: (i, 0)),
                  pl.BlockSpec((k, m), lambda i: (0, 0)),
                  pl.BlockSpec((NC * NS, _BR), lambda i: (0, i))],
        out_specs=[pl.BlockSpec((_BR, m), lambda i: (i, 0)),
                   pl.BlockSpec((_BR, 1), lambda i: (i, 0))],
        out_shape=[jax.ShapeDtypeStruct((n, m), jnp.float32),
                   jax.ShapeDtypeStruct((npd, 1), jnp.float32)],
    )(x, w, degp)


def _mid_body(agg_ref, y1_ref, dinv_ref, b1_ref, o_ref):
    dinv = dinv_ref[...]
    a = agg_ref[...]
    pre = dinv * (a[0] + a[1] + y1_ref[...]) + b1_ref[...]
    o_ref[...] = dinv * jnp.maximum(pre, 0.0)


def _mid(agg1, y1, dinv, b1):
    n, d = y1.shape
    return pl.pallas_call(
        _mid_body,
        grid=(pl.cdiv(n, _BR),),
        in_specs=[pl.BlockSpec((NC, _BR, d), lambda i: (0, i, 0)),
                  pl.BlockSpec((_BR, d), lambda i: (i, 0)),
                  pl.BlockSpec((_BR, 1), lambda i: (i, 0)),
                  pl.BlockSpec((1, d), lambda i: (0, 0))],
        out_specs=pl.BlockSpec((_BR, d), lambda i: (i, 0)),
        out_shape=jax.ShapeDtypeStruct((n, d), jnp.float32),
    )(agg1, y1, dinv, b1)


def _final_body(agg_ref, y2_ref, dinv_ref, w2_ref, b2_ref, o_ref):
    dinv = dinv_ref[...]
    a = agg_ref[...]
    t = a[0] + a[1] + y2_ref[...]
    o_ref[...] = dinv * jnp.dot(t, w2_ref[...],
                                preferred_element_type=jnp.float32) \
        + b2_ref[...]


def _final(agg2, y2, dinv, w2, b2):
    n, d = y2.shape
    m = w2.shape[1]
    return pl.pallas_call(
        _final_body,
        grid=(pl.cdiv(n, _BR),),
        in_specs=[pl.BlockSpec((NC, _BR, d), lambda i: (0, i, 0)),
                  pl.BlockSpec((_BR, d), lambda i: (i, 0)),
                  pl.BlockSpec((_BR, 1), lambda i: (i, 0)),
                  pl.BlockSpec((d, m), lambda i: (0, 0)),
                  pl.BlockSpec((1, m), lambda i: (0, 0))],
        out_specs=pl.BlockSpec((_BR, m), lambda i: (i, 0)),
        out_shape=jax.ShapeDtypeStruct((n, m), jnp.float32),
    )(agg2, y2, dinv, w2, b2)


def kernel(x, edge_index, W1, b1, W2, b2):
    n = x.shape[0]
    e = edge_index.shape[1]
    src = edge_index[0].astype(jnp.int32)
    dst = edge_index[1].astype(jnp.int32)

    degp = _make_deg_kernel(n, e)(dst)
    y1, dinv = _head(x, W1, degp)
    agg1 = _make_agg_kernel(n, e, y1.shape[1])(y1, src, dst)
    y2 = _mid(agg1, y1, dinv, b1.reshape(1, -1))
    agg2 = _make_agg_kernel(n, e, y2.shape[1])(y2, src, dst)
    return _final(agg2, y2, dinv, W2, b2.reshape(1, -1))


# speedup vs baseline: 1.0006x; 1.0006x over previous
import functools

import jax
import jax.numpy as jnp
from jax import lax
from jax.experimental import pallas as pl
from jax.experimental.pallas import tpu as pltpu
from jax.experimental.pallas import tpu_sc as plsc

NC, NS, LANES = 2, 16, 16
CH = 80
ZR = 16
DEGW = 16


def _npad(n):
    step = NS * ZR
    return ((n + step - 1) // step) * step


def _sc_mesh():
    return plsc.VectorSubcoreMesh(
        core_axis_name="c", subcore_axis_name="s",
        num_cores=NC, num_subcores=NS)


def _make_deg_kernel(n, e):
    nw = NC * NS
    e_tile = e // nw
    npd = _npad(n)

    @functools.partial(
        pl.kernel,
        out_type=jax.ShapeDtypeStruct((nw, npd), jnp.float32),
        mesh=_sc_mesh(),
        compiler_params=pltpu.CompilerParams(needs_layout_passes=False),
        scratch_types=[
            pltpu.VMEM((e_tile,), jnp.int32),
            pltpu.VMEM((npd,), jnp.float32),
        ])
    def deg_kernel(dst_hbm, out_hbm, idx_v, hist_v):
        c = lax.axis_index("c")
        s = lax.axis_index("s")
        wid = c * NS + s
        base = wid * e_tile
        pltpu.sync_copy(dst_hbm.at[pl.ds(base, e_tile)], idx_v)

        def initz(i, carry):
            hist_v[pl.ds(i * LANES, LANES)] = jnp.zeros((LANES,),
                                                        jnp.float32)
            return carry
        lax.fori_loop(0, npd // LANES, initz, None)
        ones = jnp.ones((LANES,), jnp.float32)

        def body(i, carry):
            idx = idx_v[pl.ds(i * LANES, LANES)]
            plsc.addupdate_scatter(hist_v, [idx], ones)
            return carry
        lax.fori_loop(0, e_tile // LANES, body, None)
        pltpu.sync_copy(hist_v, out_hbm.at[wid])

    return deg_kernel


_NB = 7
_ACH = 40


def _make_agg_kernel(n, e, d):
    e_sc, e_tile = e // NC, e // (NC * NS)
    nch = e_tile // _ACH
    ngroups = nch // _NB
    npd = _npad(n)
    rpt = npd // NS
    nz = rpt // ZR
    nvec = d // LANES

    @functools.partial(
        pl.kernel,
        out_type=jax.ShapeDtypeStruct((NC, npd, d), jnp.float32),
        mesh=_sc_mesh(),
        scratch_types=[
            pltpu.VMEM((e_tile,), jnp.int32),
            [pltpu.VMEM((_ACH,), jnp.int32) for _ in range(_NB)],
            [pltpu.VMEM((_ACH, d), jnp.float32) for _ in range(_NB)],
            pltpu.VMEM((ZR, d), jnp.float32),
            pltpu.VMEM_SHARED((npd, d), jnp.float32),
            pltpu.SemaphoreType.DMA((_NB,)),
            pltpu.SemaphoreType.DMA((_NB,)),
            pltpu.SemaphoreType.DMA,
        ])
    def agg_kernel(y_hbm, src_hbm, dst_hbm, out_hbm,
                   srcall_v, dring, rows, zero_v, acc_sh, dsem, gsem,
                   zsem):
        c = lax.axis_index("c")
        s = lax.axis_index("s")
        base = c * e_sc + s * e_tile

        pltpu.sync_copy(src_hbm.at[pl.ds(base, e_tile)], srcall_v)

        def issue(j, b):
            off = pl.multiple_of(base + j * _ACH, 8)
            pltpu.async_copy(dst_hbm.at[pl.ds(off, _ACH)], dring[b],
                             dsem.at[b])
            idx = srcall_v.at[pl.ds(j * _ACH, _ACH)]
            pltpu.async_copy(y_hbm.at[idx], rows[b], gsem.at[b])

        def drain_and_scatter(j, b):
            off = pl.multiple_of(base + j * _ACH, 8)
            pltpu.make_async_copy(dst_hbm.at[pl.ds(off, _ACH)], dring[b],
                                  dsem.at[b]).wait()
            idx = srcall_v.at[pl.ds(j * _ACH, _ACH)]
            pltpu.make_async_copy(y_hbm.at[idx], rows[b],
                                  gsem.at[b]).wait()
            pltpu.sync_copy(rows[b], acc_sh.at[dring[b]], add=True)

        for b in range(_NB):
            issue(b, b)

        def initz(i, carry):
            for q in range(nvec):
                zero_v[i, pl.ds(q * LANES, LANES)] = (
                    jnp.zeros((LANES,), jnp.float32))
            return carry
        lax.fori_loop(0, ZR, initz, None)

        r0 = s * rpt
        for q in range(nz):
            pltpu.async_copy(zero_v, acc_sh.at[pl.ds(r0 + q * ZR, ZR)],
                             zsem)
        for q in range(nz):
            pltpu.make_async_copy(zero_v,
                                  acc_sh.at[pl.ds(r0 + q * ZR, ZR)],
                                  zsem).wait()
        plsc.subcore_barrier()

        def group(t, carry):
            j0 = t * _NB
            for b in range(_NB):
                drain_and_scatter(j0 + b, b)
                issue(j0 + b + _NB, b)
            return carry
        lax.fori_loop(0, ngroups - 1, group, None)
        for b in range(_NB):
            drain_and_scatter((ngroups - 1) * _NB + b, b)
        for j in range(ngroups * _NB, nch):
            issue(j, 0)
            drain_and_scatter(j, 0)

        plsc.subcore_barrier()
        pltpu.sync_copy(acc_sh.at[pl.ds(r0, rpt)],
                        out_hbm.at[c, pl.ds(r0, rpt)])

    return agg_kernel


_BR = 1024


def _head_body(x_ref, w_ref, degp_ref, y_ref, dinv_ref):
    deg = jnp.sum(degp_ref[...], axis=0).reshape(-1, 1) + 1.0
    dinv = lax.rsqrt(deg)
    dinv_ref[...] = dinv
    y_ref[...] = dinv * jnp.dot(x_ref[...], w_ref[...],
                                preferred_element_type=jnp.float32)


def _head(x, w, degp):
    n, k = x.shape
    m = w.shape[1]
    npd = degp.shape[1]
    return pl.pallas_call(
        _head_body,
        grid=(pl.cdiv(npd, _BR),),
        in_specs=[pl.BlockSpec((_BR, k), lambda i: (i, 0)),
                  pl.BlockSpec((k, m), lambda i: (0, 0)),
                  pl.BlockSpec((NC * NS, _BR), lambda i: (0, i))],
        out_specs=[pl.BlockSpec((_BR, m), lambda i: (i, 0)),
                   pl.BlockSpec((_BR, 1), lambda i: (i, 0))],
        out_shape=[jax.ShapeDtypeStruct((n, m), jnp.float32),
                   jax.ShapeDtypeStruct((npd, 1), jnp.float32)],
    )(x, w, degp)


def _mid_body(agg_ref, y1_ref, dinv_ref, b1_ref, o_ref):
    dinv = dinv_ref[...]
    a = agg_ref[...]
    pre = dinv * (a[0] + a[1] + y1_ref[...]) + b1_ref[...]
    o_ref[...] = dinv * jnp.maximum(pre, 0.0)


def _mid(agg1, y1, dinv, b1):
    n, d = y1.shape
    return pl.pallas_call(
        _mid_body,
        grid=(pl.cdiv(n, _BR),),
        in_specs=[pl.BlockSpec((NC, _BR, d), lambda i: (0, i, 0)),
                  pl.BlockSpec((_BR, d), lambda i: (i, 0)),
                  pl.BlockSpec((_BR, 1), lambda i: (i, 0)),
                  pl.BlockSpec((1, d), lambda i: (0, 0))],
        out_specs=pl.BlockSpec((_BR, d), lambda i: (i, 0)),
        out_shape=jax.ShapeDtypeStruct((n, d), jnp.float32),
    )(agg1, y1, dinv, b1)


def _final_body(agg_ref, y2_ref, dinv_ref, w2_ref, b2_ref, o_ref):
    dinv = dinv_ref[...]
    a = agg_ref[...]
    t = a[0] + a[1] + y2_ref[...]
    o_ref[...] = dinv * jnp.dot(t, w2_ref[...],
                                preferred_element_type=jnp.float32) \
        + b2_ref[...]


def _final(agg2, y2, dinv, w2, b2):
    n, d = y2.shape
    m = w2.shape[1]
    return pl.pallas_call(
        _final_body,
        grid=(pl.cdiv(n, _BR),),
        in_specs=[pl.BlockSpec((NC, _BR, d), lambda i: (0, i, 0)),
                  pl.BlockSpec((_BR, d), lambda i: (i, 0)),
                  pl.BlockSpec((_BR, 1), lambda i: (i, 0)),
                  pl.BlockSpec((d, m), lambda i: (0, 0)),
                  pl.BlockSpec((1, m), lambda i: (0, 0))],
        out_specs=pl.BlockSpec((_BR, m), lambda i: (i, 0)),
        out_shape=jax.ShapeDtypeStruct((n, m), jnp.float32),
    )(agg2, y2, dinv, w2, b2)


def kernel(x, edge_index, W1, b1, W2, b2):
    n = x.shape[0]
    e = edge_index.shape[1]
    src = edge_index[0].astype(jnp.int32)
    dst = edge_index[1].astype(jnp.int32)

    degp = _make_deg_kernel(n, e)(dst)
    y1, dinv = _head(x, W1, degp)
    agg1 = _make_agg_kernel(n, e, y1.shape[1])(y1, src, dst)
    y2 = _mid(agg1, y1, dinv, b1.reshape(1, -1))
    agg2 = _make_agg_kernel(n, e, y2.shape[1])(y2, src, dst)
    return _final(agg2, y2, dinv, W2, b2.reshape(1, -1))


# speedup vs baseline: 1.0216x; 1.0210x over previous
import functools

import jax
import jax.numpy as jnp
from jax import lax
from jax.experimental import pallas as pl
from jax.experimental.pallas import tpu as pltpu
from jax.experimental.pallas import tpu_sc as plsc

NC, NS, LANES = 2, 16, 16
CH = 80
ZR = 16
DEGW = 16


def _npad(n):
    step = NS * ZR
    return ((n + step - 1) // step) * step


def _sc_mesh():
    return plsc.VectorSubcoreMesh(
        core_axis_name="c", subcore_axis_name="s",
        num_cores=NC, num_subcores=NS)


def _make_deg_kernel(n, e):
    nw = NC * NS
    e_tile = e // nw
    npd = _npad(n)

    @functools.partial(
        pl.kernel,
        out_type=jax.ShapeDtypeStruct((nw, npd), jnp.float32),
        mesh=_sc_mesh(),
        compiler_params=pltpu.CompilerParams(needs_layout_passes=False),
        scratch_types=[
            pltpu.VMEM((e_tile,), jnp.int32),
            pltpu.VMEM((npd,), jnp.float32),
        ])
    def deg_kernel(dst_hbm, out_hbm, idx_v, hist_v):
        c = lax.axis_index("c")
        s = lax.axis_index("s")
        wid = c * NS + s
        base = wid * e_tile
        pltpu.sync_copy(dst_hbm.at[pl.ds(base, e_tile)], idx_v)

        def initz(i, carry):
            hist_v[pl.ds(i * LANES, LANES)] = jnp.zeros((LANES,),
                                                        jnp.float32)
            return carry
        lax.fori_loop(0, npd // LANES, initz, None)
        ones = jnp.ones((LANES,), jnp.float32)

        def body(i, carry):
            idx = idx_v[pl.ds(i * LANES, LANES)]
            plsc.addupdate_scatter(hist_v, [idx], ones)
            return carry
        lax.fori_loop(0, e_tile // LANES, body, None)
        pltpu.sync_copy(hist_v, out_hbm.at[wid])

    return deg_kernel


_NB = 5
_ACH = 40


def _make_agg_kernel(n, e, d):
    e_sc, e_tile = e // NC, e // (NC * NS)
    nch = e_tile // _ACH
    ngroups = nch // _NB
    npd = _npad(n)
    rpt = npd // NS
    nz = rpt // ZR
    nvec = d // LANES

    @functools.partial(
        pl.kernel,
        out_type=jax.ShapeDtypeStruct((NC, npd, d), jnp.float32),
        mesh=_sc_mesh(),
        scratch_types=[
            pltpu.VMEM((e_tile,), jnp.int32),
            [pltpu.VMEM((_ACH,), jnp.int32) for _ in range(_NB)],
            [pltpu.VMEM((_ACH, d), jnp.float32) for _ in range(_NB)],
            pltpu.VMEM((ZR, d), jnp.float32),
            pltpu.VMEM_SHARED((npd, d), jnp.float32),
            pltpu.SemaphoreType.DMA((_NB,)),
            pltpu.SemaphoreType.DMA((_NB,)),
            pltpu.SemaphoreType.DMA,
        ])
    def agg_kernel(y_hbm, src_hbm, dst_hbm, out_hbm,
                   srcall_v, dring, rows, zero_v, acc_sh, dsem, gsem,
                   zsem):
        c = lax.axis_index("c")
        s = lax.axis_index("s")
        base = c * e_sc + s * e_tile

        pltpu.sync_copy(src_hbm.at[pl.ds(base, e_tile)], srcall_v)

        def issue(j, b):
            off = pl.multiple_of(base + j * _ACH, 8)
            pltpu.async_copy(dst_hbm.at[pl.ds(off, _ACH)], dring[b],
                             dsem.at[b])
            idx = srcall_v.at[pl.ds(j * _ACH, _ACH)]
            pltpu.async_copy(y_hbm.at[idx], rows[b], gsem.at[b])

        def drain_and_scatter(j, b):
            off = pl.multiple_of(base + j * _ACH, 8)
            pltpu.make_async_copy(dst_hbm.at[pl.ds(off, _ACH)], dring[b],
                                  dsem.at[b]).wait()
            idx = srcall_v.at[pl.ds(j * _ACH, _ACH)]
            pltpu.make_async_copy(y_hbm.at[idx], rows[b],
                                  gsem.at[b]).wait()
            pltpu.sync_copy(rows[b], acc_sh.at[dring[b]], add=True)

        for b in range(_NB):
            issue(b, b)

        def initz(i, carry):
            for q in range(nvec):
                zero_v[i, pl.ds(q * LANES, LANES)] = (
                    jnp.zeros((LANES,), jnp.float32))
            return carry
        lax.fori_loop(0, ZR, initz, None)

        r0 = s * rpt
        for q in range(nz):
            pltpu.async_copy(zero_v, acc_sh.at[pl.ds(r0 + q * ZR, ZR)],
                             zsem)
        for q in range(nz):
            pltpu.make_async_copy(zero_v,
                                  acc_sh.at[pl.ds(r0 + q * ZR, ZR)],
                                  zsem).wait()
        plsc.subcore_barrier()

        def group(t, carry):
            j0 = t * _NB
            for b in range(_NB):
                drain_and_scatter(j0 + b, b)
                issue(j0 + b + _NB, b)
            return carry
        lax.fori_loop(0, ngroups - 1, group, None)
        for b in range(_NB):
            drain_and_scatter((ngroups - 1) * _NB + b, b)
        for j in range(ngroups * _NB, nch):
            issue(j, 0)
            drain_and_scatter(j, 0)

        plsc.subcore_barrier()
        pltpu.sync_copy(acc_sh.at[pl.ds(r0, rpt)],
                        out_hbm.at[c, pl.ds(r0, rpt)])

    return agg_kernel


_BR = 1024


def _mm_body(x_ref, w_ref, o_ref):
    o_ref[...] = jnp.dot(x_ref[...], w_ref[...],
                         preferred_element_type=jnp.float32)


def _matmul(x, w):
    n, k = x.shape
    m = w.shape[1]
    return pl.pallas_call(
        _mm_body,
        grid=(pl.cdiv(n, _BR),),
        in_specs=[pl.BlockSpec((_BR, k), lambda i: (i, 0)),
                  pl.BlockSpec((k, m), lambda i: (0, 0))],
        out_specs=pl.BlockSpec((_BR, m), lambda i: (i, 0)),
        out_shape=jax.ShapeDtypeStruct((n, m), jnp.float32),
    )(x, w)


def _scale_body(xw_ref, degp_ref, y_ref, dinv_ref):
    deg = jnp.sum(degp_ref[...], axis=0).reshape(-1, 1) + 1.0
    dinv = lax.rsqrt(deg)
    dinv_ref[...] = dinv
    y_ref[...] = dinv * xw_ref[...]


def _scale(xw, degp):
    n, m = xw.shape
    npd = degp.shape[1]
    return pl.pallas_call(
        _scale_body,
        grid=(pl.cdiv(npd, _BR),),
        in_specs=[pl.BlockSpec((_BR, m), lambda i: (i, 0)),
                  pl.BlockSpec((NC * NS, _BR), lambda i: (0, i))],
        out_specs=[pl.BlockSpec((_BR, m), lambda i: (i, 0)),
                   pl.BlockSpec((_BR, 1), lambda i: (i, 0))],
        out_shape=[jax.ShapeDtypeStruct((n, m), jnp.float32),
                   jax.ShapeDtypeStruct((npd, 1), jnp.float32)],
    )(xw, degp)


def _mid_body(agg_ref, y1_ref, dinv_ref, b1_ref, o_ref):
    dinv = dinv_ref[...]
    a = agg_ref[...]
    pre = dinv * (a[0] + a[1] + y1_ref[...]) + b1_ref[...]
    o_ref[...] = dinv * jnp.maximum(pre, 0.0)


def _mid(agg1, y1, dinv, b1):
    n, d = y1.shape
    return pl.pallas_call(
        _mid_body,
        grid=(pl.cdiv(n, _BR),),
        in_specs=[pl.BlockSpec((NC, _BR, d), lambda i: (0, i, 0)),
                  pl.BlockSpec((_BR, d), lambda i: (i, 0)),
                  pl.BlockSpec((_BR, 1), lambda i: (i, 0)),
                  pl.BlockSpec((1, d), lambda i: (0, 0))],
        out_specs=pl.BlockSpec((_BR, d), lambda i: (i, 0)),
        out_shape=jax.ShapeDtypeStruct((n, d), jnp.float32),
    )(agg1, y1, dinv, b1)


def _final_body(agg_ref, y2_ref, dinv_ref, w2_ref, b2_ref, o_ref):
    dinv = dinv_ref[...]
    a = agg_ref[...]
    t = a[0] + a[1] + y2_ref[...]
    o_ref[...] = dinv * jnp.dot(t, w2_ref[...],
                                preferred_element_type=jnp.float32) \
        + b2_ref[...]


def _final(agg2, y2, dinv, w2, b2):
    n, d = y2.shape
    m = w2.shape[1]
    return pl.pallas_call(
        _final_body,
        grid=(pl.cdiv(n, _BR),),
        in_specs=[pl.BlockSpec((NC, _BR, d), lambda i: (0, i, 0)),
                  pl.BlockSpec((_BR, d), lambda i: (i, 0)),
                  pl.BlockSpec((_BR, 1), lambda i: (i, 0)),
                  pl.BlockSpec((d, m), lambda i: (0, 0)),
                  pl.BlockSpec((1, m), lambda i: (0, 0))],
        out_specs=pl.BlockSpec((_BR, m), lambda i: (i, 0)),
        out_shape=jax.ShapeDtypeStruct((n, m), jnp.float32),
    )(agg2, y2, dinv, w2, b2)


def kernel(x, edge_index, W1, b1, W2, b2):
    n = x.shape[0]
    e = edge_index.shape[1]
    src = edge_index[0].astype(jnp.int32)
    dst = edge_index[1].astype(jnp.int32)

    degp = _make_deg_kernel(n, e)(dst)
    xw1 = _matmul(x, W1)
    y1, dinv = _scale(xw1, degp)
    agg1 = _make_agg_kernel(n, e, y1.shape[1])(y1, src, dst)
    y2 = _mid(agg1, y1, dinv, b1.reshape(1, -1))
    agg2 = _make_agg_kernel(n, e, y2.shape[1])(y2, src, dst)
    return _final(agg2, y2, dinv, W2, b2.reshape(1, -1))


# speedup vs baseline: 1.0752x; 1.0525x over previous
import functools

import jax
import jax.numpy as jnp
from jax import lax
from jax.experimental import pallas as pl
from jax.experimental.pallas import tpu as pltpu
from jax.experimental.pallas import tpu_sc as plsc

NC, NS, LANES = 2, 16, 16
CH = 80
ZR = 16
DEGW = 16


def _npad(n):
    step = NS * ZR
    return ((n + step - 1) // step) * step


def _sc_mesh():
    return plsc.VectorSubcoreMesh(
        core_axis_name="c", subcore_axis_name="s",
        num_cores=NC, num_subcores=NS)


def _make_deg_kernel(n, e):
    nw = NC * NS
    e_tile = e // nw
    npd = _npad(n)

    @functools.partial(
        pl.kernel,
        out_type=jax.ShapeDtypeStruct((nw, npd), jnp.float32),
        mesh=_sc_mesh(),
        compiler_params=pltpu.CompilerParams(needs_layout_passes=False),
        scratch_types=[
            pltpu.VMEM((e_tile,), jnp.int32),
            pltpu.VMEM((npd,), jnp.float32),
        ])
    def deg_kernel(dst_hbm, out_hbm, idx_v, hist_v):
        c = lax.axis_index("c")
        s = lax.axis_index("s")
        wid = c * NS + s
        base = wid * e_tile
        pltpu.sync_copy(dst_hbm.at[pl.ds(base, e_tile)], idx_v)

        def initz(i, carry):
            hist_v[pl.ds(i * LANES, LANES)] = jnp.zeros((LANES,),
                                                        jnp.float32)
            return carry
        lax.fori_loop(0, npd // LANES, initz, None)
        ones = jnp.ones((LANES,), jnp.float32)

        def body(i, carry):
            idx = idx_v[pl.ds(i * LANES, LANES)]
            plsc.addupdate_scatter(hist_v, [idx], ones)
            return carry
        lax.fori_loop(0, e_tile // LANES, body, None)
        pltpu.sync_copy(hist_v, out_hbm.at[wid])

    return deg_kernel


_NB = 5
_ACH = 40


def _make_agg_kernel(n, e, d):
    e_sc, e_tile = e // NC, e // (NC * NS)
    nch = e_tile // _ACH
    ngroups = nch // _NB
    npd = _npad(n)
    rpt = npd // NS
    nz = rpt // ZR
    nvec = d // LANES

    @functools.partial(
        pl.kernel,
        out_type=jax.ShapeDtypeStruct((NC, npd, d), jnp.float32),
        mesh=_sc_mesh(),
        scratch_types=[
            pltpu.VMEM((e_tile,), jnp.int32),
            [pltpu.VMEM((_ACH,), jnp.int32) for _ in range(_NB)],
            [pltpu.VMEM((_ACH, d), jnp.float32) for _ in range(_NB)],
            pltpu.VMEM((ZR, d), jnp.float32),
            pltpu.VMEM_SHARED((npd, d), jnp.float32),
            pltpu.SemaphoreType.DMA((_NB,)),
            pltpu.SemaphoreType.DMA((_NB,)),
            pltpu.SemaphoreType.DMA,
        ])
    def agg_kernel(y_hbm, src_hbm, dst_hbm, out_hbm,
                   srcall_v, dring, rows, zero_v, acc_sh, dsem, gsem,
                   zsem):
        c = lax.axis_index("c")
        s = lax.axis_index("s")
        base = c * e_sc + s * e_tile

        pltpu.sync_copy(src_hbm.at[pl.ds(base, e_tile)], srcall_v)

        def issue(j, b):
            off = pl.multiple_of(base + j * _ACH, 8)
            pltpu.async_copy(dst_hbm.at[pl.ds(off, _ACH)], dring[b],
                             dsem.at[b])
            idx = srcall_v.at[pl.ds(j * _ACH, _ACH)]
            pltpu.async_copy(y_hbm.at[idx], rows[b], gsem.at[b])

        def drain_and_scatter(j, b):
            off = pl.multiple_of(base + j * _ACH, 8)
            pltpu.make_async_copy(dst_hbm.at[pl.ds(off, _ACH)], dring[b],
                                  dsem.at[b]).wait()
            idx = srcall_v.at[pl.ds(j * _ACH, _ACH)]
            pltpu.make_async_copy(y_hbm.at[idx], rows[b],
                                  gsem.at[b]).wait()
            pltpu.sync_copy(rows[b], acc_sh.at[dring[b]], add=True)

        for b in range(_NB):
            issue(b, b)

        def initz(i, carry):
            for q in range(nvec):
                zero_v[i, pl.ds(q * LANES, LANES)] = (
                    jnp.zeros((LANES,), jnp.float32))
            return carry
        lax.fori_loop(0, ZR, initz, None)

        r0 = s * rpt
        for q in range(nz):
            pltpu.async_copy(zero_v, acc_sh.at[pl.ds(r0 + q * ZR, ZR)],
                             zsem)
        for q in range(nz):
            pltpu.make_async_copy(zero_v,
                                  acc_sh.at[pl.ds(r0 + q * ZR, ZR)],
                                  zsem).wait()
        plsc.subcore_barrier()

        def group(t, carry):
            j0 = t * _NB
            for b in range(_NB):
                drain_and_scatter(j0 + b, b)
                issue(j0 + b + _NB, b)
            return carry
        lax.fori_loop(0, ngroups - 1, group, None)
        for b in range(_NB):
            drain_and_scatter((ngroups - 1) * _NB + b, b)
        for j in range(ngroups * _NB, nch):
            issue(j, 0)
            drain_and_scatter(j, 0)

        plsc.subcore_barrier()
        pltpu.sync_copy(acc_sh.at[pl.ds(r0, rpt)],
                        out_hbm.at[c, pl.ds(r0, rpt)])

    return agg_kernel


_BR = 1024


def _split_body(ei_ref, src_ref, dst_ref):
    ei = ei_ref[...]
    src_ref[...] = ei[0]
    dst_ref[...] = ei[1]


def _split(ei):
    e = ei.shape[1]
    return pl.pallas_call(
        _split_body,
        out_shape=[jax.ShapeDtypeStruct((e,), jnp.int32),
                   jax.ShapeDtypeStruct((e,), jnp.int32)],
    )(ei)


def _mm_body(x_ref, w_ref, o_ref):
    o_ref[...] = jnp.dot(x_ref[...], w_ref[...],
                         preferred_element_type=jnp.float32)


def _matmul(x, w):
    n, k = x.shape
    m = w.shape[1]
    return pl.pallas_call(
        _mm_body,
        grid=(pl.cdiv(n, _BR),),
        in_specs=[pl.BlockSpec((_BR, k), lambda i: (i, 0)),
                  pl.BlockSpec((k, m), lambda i: (0, 0))],
        out_specs=pl.BlockSpec((_BR, m), lambda i: (i, 0)),
        out_shape=jax.ShapeDtypeStruct((n, m), jnp.float32),
    )(x, w)


def _scale_body(xw_ref, degp_ref, y_ref, dinv_ref):
    deg = jnp.sum(degp_ref[...], axis=0).reshape(-1, 1) + 1.0
    dinv = lax.rsqrt(deg)
    dinv_ref[...] = dinv
    y_ref[...] = dinv * xw_ref[...]


def _scale(xw, degp):
    n, m = xw.shape
    npd = degp.shape[1]
    return pl.pallas_call(
        _scale_body,
        grid=(pl.cdiv(npd, _BR),),
        in_specs=[pl.BlockSpec((_BR, m), lambda i: (i, 0)),
                  pl.BlockSpec((NC * NS, _BR), lambda i: (0, i))],
        out_specs=[pl.BlockSpec((_BR, m), lambda i: (i, 0)),
                   pl.BlockSpec((_BR, 1), lambda i: (i, 0))],
        out_shape=[jax.ShapeDtypeStruct((n, m), jnp.float32),
                   jax.ShapeDtypeStruct((npd, 1), jnp.float32)],
    )(xw, degp)


def _mid_body(agg_ref, y1_ref, dinv_ref, b1_ref, o_ref):
    dinv = dinv_ref[...]
    a = agg_ref[...]
    pre = dinv * (a[0] + a[1] + y1_ref[...]) + b1_ref[...]
    o_ref[...] = dinv * jnp.maximum(pre, 0.0)


def _mid(agg1, y1, dinv, b1):
    n, d = y1.shape
    return pl.pallas_call(
        _mid_body,
        grid=(pl.cdiv(n, _BR),),
        in_specs=[pl.BlockSpec((NC, _BR, d), lambda i: (0, i, 0)),
                  pl.BlockSpec((_BR, d), lambda i: (i, 0)),
                  pl.BlockSpec((_BR, 1), lambda i: (i, 0)),
                  pl.BlockSpec((1, d), lambda i: (0, 0))],
        out_specs=pl.BlockSpec((_BR, d), lambda i: (i, 0)),
        out_shape=jax.ShapeDtypeStruct((n, d), jnp.float32),
    )(agg1, y1, dinv, b1)


def _final_body(agg_ref, y2_ref, dinv_ref, w2_ref, b2_ref, o_ref):
    dinv = dinv_ref[...]
    a = agg_ref[...]
    t = a[0] + a[1] + y2_ref[...]
    o_ref[...] = dinv * jnp.dot(t, w2_ref[...],
                                preferred_element_type=jnp.float32) \
        + b2_ref[...]


def _final(agg2, y2, dinv, w2, b2):
    n, d = y2.shape
    m = w2.shape[1]
    return pl.pallas_call(
        _final_body,
        grid=(pl.cdiv(n, _BR),),
        in_specs=[pl.BlockSpec((NC, _BR, d), lambda i: (0, i, 0)),
                  pl.BlockSpec((_BR, d), lambda i: (i, 0)),
                  pl.BlockSpec((_BR, 1), lambda i: (i, 0)),
                  pl.BlockSpec((d, m), lambda i: (0, 0)),
                  pl.BlockSpec((1, m), lambda i: (0, 0))],
        out_specs=pl.BlockSpec((_BR, m), lambda i: (i, 0)),
        out_shape=jax.ShapeDtypeStruct((n, m), jnp.float32),
    )(agg2, y2, dinv, w2, b2)


def kernel(x, edge_index, W1, b1, W2, b2):
    n = x.shape[0]
    e = edge_index.shape[1]
    src, dst = _split(edge_index.astype(jnp.int32))

    degp = _make_deg_kernel(n, e)(dst)
    xw1 = _matmul(x, W1)
    y1, dinv = _scale(xw1, degp)
    agg1 = _make_agg_kernel(n, e, y1.shape[1])(y1, src, dst)
    y2 = _mid(agg1, y1, dinv, b1.reshape(1, -1))
    agg2 = _make_agg_kernel(n, e, y2.shape[1])(y2, src, dst)
    return _final(agg2, y2, dinv, W2, b2.reshape(1, -1))
